# baseline (speedup 1.0000x reference)
import functools

import jax
import jax.numpy as jnp
from jax import lax
from jax.experimental import pallas as pl
from jax.experimental.pallas import tpu as pltpu
from jax.experimental.pallas import tpu_sc as plsc

B = 4096
F = 26
V = 100000
K = 32
D = 13

NC = 2
NS = 16
NW = NC * NS
BPW = B // NW
NIDX = BPW * F
CHUNK = 104
NCH = NIDX // CHUNK
NPASS = 2
CPP = NCH // NPASS
RPP = CPP * CHUNK
BPP = RPP // F

_mesh = plsc.VectorSubcoreMesh(core_axis_name="c", subcore_axis_name="s")

_SCRATCH = [
    pltpu.VMEM((NCH, CHUNK), jnp.int32),
    pltpu.VMEM((RPP, K), jnp.float32),
    pltpu.VMEM((NIDX,), jnp.float32),
    pltpu.VMEM((BPW, 16), jnp.float32),
    pltpu.VMEM((48,), jnp.float32),
    pltpu.VMEM((BPW, 16), jnp.float32),
    pltpu.VMEM((BPW,), jnp.float32),
    pltpu.SemaphoreType.DMA,
]


def _fm_body(idx_hbm, xd_hbm, lin_hbm, fm_hbm, wb_hbm, out_hbm,
             idx_v, rows_v, lin_v, xd_v, wb_v, part_v, out_v, sem):
    wid = lax.axis_index("s") * NC + lax.axis_index("c")
    base = wid * BPW

    pltpu.sync_copy(idx_hbm.at[wid], idx_v)
    pltpu.sync_copy(xd_hbm.at[wid], xd_v)
    pltpu.sync_copy(wb_hbm, wb_v)

    ii = lax.iota(jnp.int32, 16)
    wvec = wb_v[pl.ds(0, 16)]
    hi_mask = wb_v[pl.ds(16, 16)]
    bvec = wb_v[pl.ds(32, 16)]
    hi_off = jnp.minimum(16 + ii, F - 1)

    for p in range(NPASS):
        cps = []
        for l in range(CPP):
            j = p * CPP + l
            cps.append(pltpu.async_copy(
                fm_hbm.at[idx_v.at[j]],
                rows_v.at[pl.ds(l * CHUNK, CHUNK)], sem))
            cps.append(pltpu.async_copy(
                lin_hbm.at[idx_v.at[j]],
                lin_v.at[pl.ds(j * CHUNK, CHUNK)], sem))
        for cp in cps:
            cp.wait()

        def row_body(b, carry):
            rbase = b * F
            g = p * BPP + b
            r0 = rows_v[rbase, pl.ds(0, 16)]
            r1 = rows_v[rbase, pl.ds(16, 16)]
            s_lo, s_hi = r0, r1
            ss_lo, ss_hi = r0 * r0, r1 * r1
            for f in range(1, F):
                r0 = rows_v[rbase + f, pl.ds(0, 16)]
                r1 = rows_v[rbase + f, pl.ds(16, 16)]
                s_lo = s_lo + r0
                s_hi = s_hi + r1
                ss_lo = ss_lo + r0 * r0
                ss_hi = ss_hi + r1 * r1
            t = 0.5 * (s_lo * s_lo + s_hi * s_hi - ss_lo - ss_hi)
            gbase = g * F
            l0 = plsc.load_gather(lin_v, [gbase + ii])
            l1 = plsc.load_gather(lin_v, [gbase + hi_off]) * hi_mask
            xrow = xd_v[g, pl.ds(0, 16)]
            part_v[g, pl.ds(0, 16)] = t + l0 + l1 + xrow * wvec + bvec
            return carry

        lax.fori_loop(0, BPP, row_body, 0)

    for c in range(BPW // 16):
        ov = jnp.zeros((16,), jnp.float32)
        rows16 = c * 16 + ii
        for l in range(16):
            ov = ov + plsc.load_gather(
                part_v, [rows16, jnp.full((16,), l, jnp.int32)])
        ov = 1.0 / (1.0 + jnp.exp(-ov))
        out_v[pl.ds(c * 16, 16)] = ov

    pltpu.sync_copy(out_v, out_hbm.at[pl.ds(base, BPW)])


_fm_sc_kernel = functools.partial(
    pl.kernel,
    out_type=jax.ShapeDtypeStruct((B,), jnp.float32),
    mesh=_mesh,
    compiler_params=pltpu.CompilerParams(use_tc_tiling_on_sc=False,
                                         needs_layout_passes=False),
    scratch_types=_SCRATCH,
)(_fm_body)


def kernel(X_sparse, X_dense, lin_tables, fm_tables, W_dense, b_dense):
    flat_idx = (X_sparse.astype(jnp.int32)
                + (jnp.arange(F, dtype=jnp.int32) * V)[None, :])
    idx3 = flat_idx.reshape(NW, NCH, CHUNK)
    xd = jnp.pad(X_dense, ((0, 0), (0, 16 - D))).reshape(NW, BPW, 16)
    lin_flat = lin_tables.reshape(F * V)
    fm_flat = fm_tables.reshape(F * V, K)
    lane_mask = (jnp.arange(16) < (F - 16)).astype(jnp.float32)
    bias_vec = jnp.zeros((16,), jnp.float32).at[0].set(b_dense[0])
    wb = jnp.concatenate([W_dense[:, 0], jnp.zeros((3,), jnp.float32),
                          lane_mask, bias_vec])
    return _fm_sc_kernel(idx3, xd, lin_flat, fm_flat, wb)
